# baseline (speedup 1.0000x reference)
import functools

import jax
import jax.numpy as jnp
import numpy as np
from jax import lax
from jax.experimental import pallas as pl
from jax.experimental.pallas import tpu as pltpu
from jax.experimental.pallas import tpu_sc as plsc

N_POINTS = 262144
N_LEVELS = 16
N_FEATS = 2
HASHMAP_SIZE = 1 << 19
MASK = HASHMAP_SIZE - 1
P1 = -1640531535
P2 = 805459861

_SCALES = [np.float32(16.0 * (1.5 ** l) - 1.0) for l in range(N_LEVELS)]
_RES = [int(np.ceil(16.0 * (1.5 ** l) - 1.0)) + 1 for l in range(N_LEVELS)]
_USE_HASH = [(r ** 3) > HASHMAP_SIZE for r in _RES]

NC = 2
NS = 16
NW = NC * NS
PTS_PER_W = N_POINTS // NW
C = 128
NCHUNK = PTS_PER_W // C
G = C // 16


def _body(x_hbm, grid_hbm, out_hbm, xbuf, idxbuf, featbuf, wbuf, outbuf, sem):
    wid = lax.axis_index("s") * NC + lax.axis_index("c")
    base_pt = wid * PTS_PER_W
    pltpu.sync_copy(x_hbm.at[pl.ds(base_pt, PTS_PER_W)], xbuf)

    iota16 = lax.iota(jnp.int32, 16)
    zeros16 = jnp.zeros((16,), jnp.float32)

    def chunk_body(ch, carry):
        off = ch * C

        def do_level(l):
            scale = _SCALES[l]
            lvl_off = l * HASHMAP_SIZE

            def pass1(g, c1):
                pvec = off + g * 16 + iota16
                spos = (g * 16 + iota16) * 4
                xs = [plsc.load_gather(xbuf, [pvec, jnp.full((16,), d, jnp.int32)])
                      for d in range(3)]
                pos = [x * scale + jnp.float32(0.5) for x in xs]
                pi = [q.astype(jnp.int32) for q in pos]
                for d in range(3):
                    wbuf[d, pl.ds(g * 16, 16)] = pos[d] - pi[d].astype(jnp.float32)
                if _USE_HASH[l]:
                    t0 = [pi[0], pi[0] + 1]
                    m1 = pi[1] * P1
                    m2 = pi[2] * P2
                    t1 = [m1, m1 + P1]
                    t2 = [m2, m2 + P2]
                    for c in range(8):
                        h = t0[c & 1] ^ t1[(c >> 1) & 1] ^ t2[(c >> 2) & 1]
                        idx = ((h & MASK) + lvl_off) * 4
                        plsc.store_scatter(
                            idxbuf, [jnp.full((16,), c, jnp.int32), spos], idx)
                else:
                    res = _RES[l]
                    m = res - 1
                    t0 = [jnp.minimum(pi[0], m), jnp.minimum(pi[0] + 1, m)]
                    t1 = [jnp.minimum(pi[1], m) * res,
                          jnp.minimum(pi[1] + 1, m) * res]
                    t2 = [jnp.minimum(pi[2], m) * (res * res),
                          jnp.minimum(pi[2] + 1, m) * (res * res)]
                    for c in range(8):
                        idx = t0[c & 1] + t1[(c >> 1) & 1] + t2[(c >> 2) & 1]
                        idx = (idx + lvl_off) * 4
                        plsc.store_scatter(
                            idxbuf, [jnp.full((16,), c, jnp.int32), spos], idx)
                return c1

            lax.fori_loop(0, G, pass1, 0)

            copies = [pltpu.async_copy(
                grid_hbm.at[idxbuf.at[c]], featbuf.at[c], sem)
                for c in range(8)]
            for cp in copies:
                cp.wait()

            def pass2(g, c2):
                pvec = g * 16 + iota16
                w0 = wbuf[0, pl.ds(g * 16, 16)]
                w1 = wbuf[1, pl.ds(g * 16, 16)]
                w2 = wbuf[2, pl.ds(g * 16, 16)]
                one = jnp.float32(1.0)
                u0, u1, u2 = one - w0, one - w1, one - w2
                a = [u0 * u1, w0 * u1, u0 * w1, w0 * w1]
                wts = [a[0] * u2, a[1] * u2, a[2] * u2, a[3] * u2,
                       a[0] * w2, a[1] * w2, a[2] * w2, a[3] * w2]
                acc0 = zeros16
                acc1 = zeros16
                for c in range(8):
                    cv = jnp.full((16,), c, jnp.int32)
                    f0 = plsc.load_gather(
                        featbuf, [cv, pvec, jnp.zeros((16,), jnp.int32)])
                    f1 = plsc.load_gather(
                        featbuf, [cv, pvec, jnp.ones((16,), jnp.int32)])
                    acc0 = acc0 + wts[c] * f0
                    acc1 = acc1 + wts[c] * f1
                plsc.store_scatter(
                    outbuf, [pvec, jnp.full((16,), 2 * l, jnp.int32)], acc0)
                plsc.store_scatter(
                    outbuf, [pvec, jnp.full((16,), 2 * l + 1, jnp.int32)], acc1)
                return c2

            lax.fori_loop(0, G, pass2, 0)

        for l in range(N_LEVELS):
            do_level(l)

        pltpu.sync_copy(outbuf, out_hbm.at[pl.ds(base_pt + off, C)])
        return carry

    lax.fori_loop(0, NCHUNK, chunk_body, 0)


_encode_sc = functools.partial(
    pl.kernel,
    mesh=plsc.VectorSubcoreMesh(core_axis_name="c", subcore_axis_name="s"),
    compiler_params=pltpu.CompilerParams(
        needs_layout_passes=False, use_tc_tiling_on_sc=False),
    out_type=jax.ShapeDtypeStruct((N_POINTS, N_LEVELS * N_FEATS), jnp.float32),
    scratch_types=[
        pltpu.VMEM((PTS_PER_W, 3), jnp.float32),
        pltpu.VMEM((8, 4 * C), jnp.int32),
        pltpu.VMEM((8, 4 * C, N_FEATS), jnp.float32),
        pltpu.VMEM((3, C), jnp.float32),
        pltpu.VMEM((C, N_LEVELS * N_FEATS), jnp.float32),
        pltpu.SemaphoreType.DMA,
    ],
)(_body)


def kernel(x, grid):
    grid2 = grid.reshape(N_LEVELS * HASHMAP_SIZE, N_FEATS)
    return _encode_sc(x, grid2)
